# baseline (speedup 1.0000x reference)
import functools

import jax
import jax.numpy as jnp
from jax import lax
from jax.experimental import pallas as pl
from jax.experimental.pallas import tpu as pltpu
from jax.experimental.pallas import tpu_sc as plsc

NUM_EXAMP = 100000
NUM_CLASS = 128
BATCH = 16384
LAM = 2.0
W_EMA = 0.7
TEMP = 0.33
WARMUP = 30

NC = 2
NS = 16
NW = NC * NS
RPT = NUM_EXAMP // NW
TS = 3136
LCAP = TS + 128
CHUNK = 128
BPW = BATCH // NW
BB = 512

_mesh = plsc.VectorSubcoreMesh(core_axis_name="c", subcore_axis_name="s")
_sc_params = pltpu.CompilerParams(needs_layout_passes=False)


PCH = 64
NPC = BPW // PCH
PPE = BATCH // NPC


@functools.partial(
    pl.kernel,
    out_type=(
        jax.ShapeDtypeStruct((BATCH, NUM_CLASS), jnp.float32),
        jax.ShapeDtypeStruct((NW, LCAP), jnp.int32),
        jax.ShapeDtypeStruct((NW, LCAP), jnp.int32),
        jax.ShapeDtypeStruct((NW, 16), jnp.int32),
    ),
    mesh=_mesh,
    compiler_params=_sc_params,
    scratch_types=[
        pltpu.VMEM((BATCH,), jnp.int32),
        pltpu.VMEM((16 * TS,), jnp.int32),
        pltpu.VMEM((LCAP,), jnp.int32),
        pltpu.VMEM((LCAP,), jnp.int32),
        pltpu.VMEM((16,), jnp.int32),
        pltpu.VMEM((PCH, NUM_CLASS), jnp.float32),
        pltpu.VMEM((PCH, NUM_CLASS), jnp.float32),
        pltpu.VMEM((PCH, NUM_CLASS), jnp.float32),
        pltpu.VMEM((PCH, NUM_CLASS), jnp.float32),
        pltpu.SemaphoreType.DMA,
        pltpu.SemaphoreType.DMA,
        pltpu.SemaphoreType.DMA,
        pltpu.SemaphoreType.DMA,
        pltpu.SemaphoreType.DMA,
        pltpu.SemaphoreType.DMA,
        pltpu.SemaphoreType.DMA,
    ],
)
def _prep(idx_hbm, ema_hbm, o0_hbm, new_hbm, dsts_hbm, srcs_hbm, cnts_hbm,
          idxv, tab, dstl, srcl, cntv, rb0, rb1, ob0, ob1,
          isem, g0, g1, p0, p1, w0, w1):
  wid = lax.axis_index("s") * NC + lax.axis_index("c")
  lo = wid * RPT
  base = wid * BPW
  rbs, obs = (rb0, rb1), (ob0, ob1)
  gs, ps, ws = (g0, g1), (p0, p1), (w0, w1)

  pltpu.async_copy(idx_hbm, idxv, isem)

  lanes = lax.iota(jnp.int32, 16)
  neg1 = jnp.full((16,), -1, jnp.int32)

  @plsc.parallel_loop(0, TS, unroll=4)
  def _(i):
    tab[pl.ds(i * 16, 16)] = neg1

  pltpu.make_async_copy(idx_hbm, idxv, isem).wait()

  def issue(c, s):
    off = base + c * PCH
    pltpu.async_copy(ema_hbm.at[idxv.at[pl.ds(off, PCH)]], rbs[s], gs[s])
    pltpu.async_copy(o0_hbm.at[pl.ds(off, PCH)], obs[s], ps[s])

  issue(0, 0)
  issue(1, 1)

  laneoff = lanes * TS - lo

  def p1_piece(piece):
    def body(k, carry):
      for u in range(4):
        off = piece * PPE + k * 64 + u * 16
        v = idxv[pl.ds(off, 16)]
        pos = lanes + off
        m = (v >= lo) & (v < lo + RPT)
        plsc.store_scatter(tab, [v + laneoff], pos, mask=m)
      return carry

    lax.fori_loop(0, PPE // 64, body, 0)

  for c in range(NPC):
    s = c % 2
    pltpu.make_async_copy(ema_hbm.at[idxv.at[pl.ds(0, PCH)]], rbs[s],
                          gs[s]).wait()
    pltpu.make_async_copy(o0_hbm.at[pl.ds(0, PCH)], obs[s], ps[s]).wait()

    def blend(r, carry, rb=rbs[s], ob=obs[s]):
      for q in range(NUM_CLASS // 16):
        sl = pl.ds(q * 16, 16)
        rb[r, sl] = W_EMA * rb[r, sl] + (1.0 - W_EMA) * ob[r, sl]
      return carry

    lax.fori_loop(0, PCH, blend, 0)
    pltpu.async_copy(rbs[s], new_hbm.at[pl.ds(base + c * PCH, PCH)], ws[s])
    p1_piece(c)
    if c + 2 < NPC:
      pltpu.make_async_copy(rbs[s], new_hbm.at[pl.ds(0, PCH)], ws[s]).wait()
      issue(c + 2, s)
  pltpu.make_async_copy(rb0, new_hbm.at[pl.ds(0, PCH)], ws[0]).wait()
  pltpu.make_async_copy(rb1, new_hbm.at[pl.ds(0, PCH)], ws[1]).wait()

  @plsc.parallel_loop(0, TS // 16, unroll=2, carry=jnp.int32(0))
  def cnt(j, c_in):
    acc = tab[pl.ds(j * 16, 16)]
    for l in range(1, 16):
      acc = jnp.maximum(acc, tab[pl.ds(l * TS + j * 16, 16)])
    m = acc >= 0
    rows = lanes + (j * 16 + lo)
    plsc.store_compressed(dstl.at[pl.ds(c_in, 16)], rows, mask=m)
    plsc.store_compressed(srcl.at[pl.ds(c_in, 16)], acc, mask=m)
    return c_in + jnp.sum(jnp.where(m, 1, 0))

  @pl.when(cnt > 0)
  def _():
    dlast = dstl[pl.ds(cnt - 1, 16)]
    slast = srcl[pl.ds(cnt - 1, 16)]
    dpad = jnp.full((16,), dlast[0], jnp.int32)
    spad = jnp.full((16,), slast[0], jnp.int32)
    for q in range(CHUNK // 16):
      dstl[pl.ds(cnt + q * 16, 16)] = dpad
      srcl[pl.ds(cnt + q * 16, 16)] = spad

  cntv[...] = jnp.full((16,), cnt, jnp.int32)
  pltpu.sync_copy(dstl, dsts_hbm.at[wid])
  pltpu.sync_copy(srcl, srcs_hbm.at[wid])
  pltpu.sync_copy(cntv, cnts_hbm.at[wid])


@functools.partial(
    pl.kernel,
    out_type=(),
    mesh=_mesh,
    compiler_params=_sc_params,
    scratch_types=[
        pltpu.VMEM((LCAP,), jnp.int32),
        pltpu.VMEM((LCAP,), jnp.int32),
        pltpu.VMEM((16,), jnp.int32),
        pltpu.VMEM((CHUNK,), jnp.int32),
        pltpu.VMEM((CHUNK,), jnp.int32),
        pltpu.VMEM((CHUNK,), jnp.int32),
        pltpu.VMEM((CHUNK,), jnp.int32),
        pltpu.VMEM((CHUNK, NUM_CLASS), jnp.float32),
        pltpu.VMEM((CHUNK, NUM_CLASS), jnp.float32),
        pltpu.SemaphoreType.DMA,
        pltpu.SemaphoreType.DMA,
        pltpu.SemaphoreType.DMA,
    ],
)
def _scatter_exec(dsts_hbm, srcs_hbm, cnts_hbm, new_hbm, ema_ref,
                  dstl, srcl, cntv, db0, db1, sb0, sb1,
                  rb0, rb1, g0, g1, ssem):
  wid = lax.axis_index("s") * NC + lax.axis_index("c")
  pltpu.async_copy(cnts_hbm.at[wid], cntv, g0)
  pltpu.async_copy(dsts_hbm.at[wid], dstl, g1)
  pltpu.async_copy(srcs_hbm.at[wid], srcl, ssem)
  pltpu.make_async_copy(cnts_hbm.at[wid], cntv, g0).wait()
  pltpu.make_async_copy(dsts_hbm.at[wid], dstl, g1).wait()
  pltpu.make_async_copy(srcs_hbm.at[wid], srcl, ssem).wait()
  cnt = cntv[pl.ds(0, 16)][0]

  @pl.when(cnt > 0)
  def _():
    nch = (cnt + CHUNK - 1) // CHUNK
    dbs, sbs, rbs, gs = (db0, db1), (sb0, sb1), (rb0, rb1), (g0, g1)

    def prep_and_gather(c, s):
      for q in range(CHUNK // 16):
        dbs[s][pl.ds(q * 16, 16)] = dstl[pl.ds(c * CHUNK + q * 16, 16)]
        sbs[s][pl.ds(q * 16, 16)] = srcl[pl.ds(c * CHUNK + q * 16, 16)]
      pltpu.async_copy(new_hbm.at[sbs[s]], rbs[s], gs[s])

    prep_and_gather(0, 0)

    def outer(i, carry):
      for s in range(2):
        c = i * 2 + s

        @pl.when(c < nch)
        def _():
          pltpu.make_async_copy(new_hbm.at[sbs[s]], rbs[s], gs[s]).wait()

          @pl.when(c > 0)
          def _():
            pltpu.make_async_copy(rbs[s], ema_ref.at[dbs[s]], ssem).wait()

          @pl.when(c + 1 < nch)
          def _():
            prep_and_gather(c + 1, 1 - s)

          pltpu.async_copy(rbs[s], ema_ref.at[dbs[s]], ssem)

      return carry

    lax.fori_loop(0, (nch + 1) // 2, outer, 0)
    pltpu.make_async_copy(rb0, ema_ref.at[db0], ssem).wait()


def _dense_body(w_ref, t_ref, o0_ref, o1_ref, new_ref, ema_ref,
                emac_ref, tot_ref, acc):
  pid = pl.program_id(0)
  emac_ref[...] = ema_ref[...]
  o0 = o0_ref[...]
  o1 = o1_ref[...]
  new = new_ref[...]
  t = t_ref[0, 0, :]
  mask = (lax.broadcasted_iota(jnp.int32, o0.shape, 1) == t[:, None]).astype(
      jnp.float32)

  m0 = jnp.max(o0, axis=1, keepdims=True)
  lse0 = jnp.log(jnp.sum(jnp.exp(o0 - m0), axis=1)) + m0[:, 0]
  logp0t = jnp.sum(o0 * mask, axis=1) - lse0

  z = new * (1.0 / TEMP)
  mz = jnp.max(z, axis=1, keepdims=True)
  ez = jnp.exp(z - mz)
  alpha = jnp.sum(ez * mask, axis=1) / jnp.sum(ez, axis=1)
  w = w_ref[0]
  alphaw = w * alpha + (1.0 - w)

  m1 = jnp.max(o1, axis=1, keepdims=True)
  lse1 = jnp.log(jnp.sum(jnp.exp(o1 - m1), axis=1)) + m1[:, 0]
  t1 = jnp.sum(o1 * mask, axis=1) - lse1
  s1 = jnp.sum(o1, axis=1) - NUM_CLASS * lse1
  mterm = alphaw * t1 + (1.0 - alphaw) * (s1 - t1) * (1.0 / (NUM_CLASS - 1))

  part = jnp.sum(logp0t + LAM * mterm)

  @pl.when(pid == 0)
  def _():
    acc[0] = 0.0

  acc[0] += part

  @pl.when(pid == pl.num_programs(0) - 1)
  def _():
    tot_ref[0, 0] = -acc[0] / BATCH


_EB = 409600


def _dense(w, targets3d, outputs_0, outputs_1, new_rows, ema_flat):
  return pl.pallas_call(
      _dense_body,
      grid=(BATCH // BB,),
      in_specs=[
          pl.BlockSpec(memory_space=pltpu.SMEM),
          pl.BlockSpec((1, 1, BB), lambda i: (i, 0, 0)),
          pl.BlockSpec((BB, NUM_CLASS), lambda i: (i, 0)),
          pl.BlockSpec((BB, NUM_CLASS), lambda i: (i, 0)),
          pl.BlockSpec((BB, NUM_CLASS), lambda i: (i, 0)),
          pl.BlockSpec((_EB,), lambda i: (i,)),
      ],
      out_specs=[
          pl.BlockSpec((_EB,), lambda i: (i,)),
          pl.BlockSpec(memory_space=pltpu.SMEM),
      ],
      out_shape=[
          jax.ShapeDtypeStruct((NUM_EXAMP * NUM_CLASS,), jnp.float32),
          jax.ShapeDtypeStruct((1, 1), jnp.float32),
      ],
      scratch_shapes=[pltpu.SMEM((1,), jnp.float32)],
  )(w, targets3d, outputs_0, outputs_1, new_rows, ema_flat)


def kernel(outputs_0, outputs_1, targets, epoch, indexs, ema):
  w = jnp.minimum(jnp.float32(1.0),
                  jnp.asarray(epoch, jnp.float32) / WARMUP).reshape(1)
  targets3d = targets.reshape(BATCH // BB, 1, BB)

  new_rows, dsts, srcs, cnts = _prep(indexs, ema, outputs_0)
  ema_copy_flat, tot = _dense(w, targets3d, outputs_0, outputs_1, new_rows,
                              ema.reshape(-1))

  ema_ref = jax.new_ref(ema_copy_flat.reshape(NUM_EXAMP, NUM_CLASS))
  _scatter_exec(dsts, srcs, cnts, new_rows, ema_ref)
  ema_new = ema_ref[...]

  return (tot[0, 0], ema_new)


# speedup vs baseline: 1.0721x; 1.0721x over previous
import functools

import jax
import jax.numpy as jnp
from jax import lax
from jax.experimental import pallas as pl
from jax.experimental.pallas import tpu as pltpu
from jax.experimental.pallas import tpu_sc as plsc

NUM_EXAMP = 100000
NUM_CLASS = 128
BATCH = 16384
LAM = 2.0
W_EMA = 0.7
TEMP = 0.33
WARMUP = 30

NC = 2
NS = 16
NW = NC * NS
RPT = NUM_EXAMP // NW
TS = 3136
LCAP = TS + 128
CHUNK = 128
BPW = BATCH // NW
BB = 512

_mesh = plsc.VectorSubcoreMesh(core_axis_name="c", subcore_axis_name="s")
_sc_params = pltpu.CompilerParams(needs_layout_passes=False)


@functools.partial(
    pl.kernel,
    out_type=jax.ShapeDtypeStruct((BATCH, NUM_CLASS), jnp.float32),
    mesh=_mesh,
    compiler_params=_sc_params,
    scratch_types=[
        pltpu.VMEM((CHUNK,), jnp.int32),
        pltpu.VMEM((CHUNK,), jnp.int32),
        pltpu.VMEM((CHUNK,), jnp.int32),
        pltpu.VMEM((CHUNK, NUM_CLASS), jnp.float32),
        pltpu.VMEM((CHUNK, NUM_CLASS), jnp.float32),
        pltpu.VMEM((CHUNK, NUM_CLASS), jnp.float32),
        pltpu.VMEM((CHUNK, NUM_CLASS), jnp.float32),
        pltpu.VMEM((CHUNK, NUM_CLASS), jnp.float32),
        pltpu.VMEM((CHUNK, NUM_CLASS), jnp.float32),
        pltpu.SemaphoreType.DMA,
        pltpu.SemaphoreType.DMA,
        pltpu.SemaphoreType.DMA,
        pltpu.SemaphoreType.DMA,
        pltpu.SemaphoreType.DMA,
        pltpu.SemaphoreType.DMA,
        pltpu.SemaphoreType.DMA,
        pltpu.SemaphoreType.DMA,
        pltpu.SemaphoreType.DMA,
    ],
)
def _gather_blend(idx_hbm, ema_hbm, o0_hbm, new_hbm,
                  ib0, ib1, ib2, rb0, rb1, rb2, ob0, ob1, ob2,
                  g0, g1, g2, p0, p1, p2, w0, w1, w2):
  wid = lax.axis_index("s") * NC + lax.axis_index("c")
  base = wid * BPW
  ibs, rbs, obs = (ib0, ib1, ib2), (rb0, rb1, rb2), (ob0, ob1, ob2)
  gs, ps, ws = (g0, g1, g2), (p0, p1, p2), (w0, w1, w2)

  def idx_of(c):
    return base + c * CHUNK

  def issue(c, s):
    pltpu.sync_copy(idx_hbm.at[pl.ds(idx_of(c), CHUNK)], ibs[s])
    pltpu.async_copy(ema_hbm.at[ibs[s]], rbs[s], gs[s])
    pltpu.async_copy(o0_hbm.at[pl.ds(idx_of(c), CHUNK)], obs[s], ps[s])

  nch = BPW // CHUNK
  issue(0, 0)
  issue(1, 1)
  issue(2, 2)
  for c in range(nch):
    s = c % 3
    pltpu.make_async_copy(ema_hbm.at[ibs[s]], rbs[s], gs[s]).wait()
    pltpu.make_async_copy(o0_hbm.at[pl.ds(0, CHUNK)], obs[s], ps[s]).wait()

    def blend(r, carry, rb=rbs[s], ob=obs[s]):
      for q in range(NUM_CLASS // 16):
        sl = pl.ds(q * 16, 16)
        rb[r, sl] = W_EMA * rb[r, sl] + (1.0 - W_EMA) * ob[r, sl]
      return carry

    lax.fori_loop(0, CHUNK, blend, 0)
    pltpu.async_copy(rbs[s], new_hbm.at[pl.ds(idx_of(c), CHUNK)], ws[s])
    if c + 3 < nch:
      pltpu.make_async_copy(rbs[s], new_hbm.at[pl.ds(0, CHUNK)],
                            ws[s]).wait()
      issue(c + 3, s)
  for s in range(min(3, nch)):
    pltpu.make_async_copy(rbs[s], new_hbm.at[pl.ds(0, CHUNK)], ws[s]).wait()


@functools.partial(
    pl.kernel,
    out_type=(
        jax.ShapeDtypeStruct((NW, LCAP), jnp.int32),
        jax.ShapeDtypeStruct((NW, LCAP), jnp.int32),
        jax.ShapeDtypeStruct((NW, 16), jnp.int32),
    ),
    mesh=_mesh,
    compiler_params=_sc_params,
    scratch_types=[
        pltpu.VMEM((BATCH,), jnp.int32),
        pltpu.VMEM((16 * TS,), jnp.int32),
        pltpu.VMEM((LCAP,), jnp.int32),
        pltpu.VMEM((LCAP,), jnp.int32),
        pltpu.VMEM((16,), jnp.int32),
        pltpu.SemaphoreType.DMA,
    ],
)
def _plan(idx_hbm, dsts_hbm, srcs_hbm, cnts_hbm, idxv, tab, dstl, srcl, cntv,
          isem):
  wid = lax.axis_index("s") * NC + lax.axis_index("c")
  lo = wid * RPT
  pltpu.async_copy(idx_hbm, idxv, isem)

  lanes = lax.iota(jnp.int32, 16)
  neg1 = jnp.full((16,), -1, jnp.int32)

  @plsc.parallel_loop(0, TS, unroll=4)
  def _(i):
    tab[pl.ds(i * 16, 16)] = neg1

  pltpu.make_async_copy(idx_hbm, idxv, isem).wait()
  laneoff = lanes * TS - lo

  def p1_body(k, carry):
    for u in range(4):
      off = k * 64 + u * 16
      v = idxv[pl.ds(off, 16)]
      pos = lanes + off
      m = (v >= lo) & (v < lo + RPT)
      plsc.store_scatter(tab, [v + laneoff], pos, mask=m)
    return carry

  lax.fori_loop(0, BATCH // 64, p1_body, 0)

  @plsc.parallel_loop(0, TS // 16, unroll=2, carry=jnp.int32(0))
  def cnt(j, c_in):
    acc = tab[pl.ds(j * 16, 16)]
    for l in range(1, 16):
      acc = jnp.maximum(acc, tab[pl.ds(l * TS + j * 16, 16)])
    m = acc >= 0
    rows = lanes + (j * 16 + lo)
    plsc.store_compressed(dstl.at[pl.ds(c_in, 16)], rows, mask=m)
    plsc.store_compressed(srcl.at[pl.ds(c_in, 16)], acc, mask=m)
    return c_in + jnp.sum(jnp.where(m, 1, 0))

  @pl.when(cnt > 0)
  def _():
    dlast = dstl[pl.ds(cnt - 1, 16)]
    slast = srcl[pl.ds(cnt - 1, 16)]
    dpad = jnp.full((16,), dlast[0], jnp.int32)
    spad = jnp.full((16,), slast[0], jnp.int32)
    for q in range(CHUNK // 16):
      dstl[pl.ds(cnt + q * 16, 16)] = dpad
      srcl[pl.ds(cnt + q * 16, 16)] = spad

  cntv[...] = jnp.full((16,), cnt, jnp.int32)
  pltpu.sync_copy(dstl, dsts_hbm.at[wid])
  pltpu.sync_copy(srcl, srcs_hbm.at[wid])
  pltpu.sync_copy(cntv, cnts_hbm.at[wid])


@functools.partial(
    pl.kernel,
    out_type=(),
    mesh=_mesh,
    compiler_params=_sc_params,
    scratch_types=[
        pltpu.VMEM((LCAP,), jnp.int32),
        pltpu.VMEM((LCAP,), jnp.int32),
        pltpu.VMEM((16,), jnp.int32),
        pltpu.VMEM((CHUNK,), jnp.int32),
        pltpu.VMEM((CHUNK,), jnp.int32),
        pltpu.VMEM((CHUNK,), jnp.int32),
        pltpu.VMEM((CHUNK,), jnp.int32),
        pltpu.VMEM((CHUNK, NUM_CLASS), jnp.float32),
        pltpu.VMEM((CHUNK, NUM_CLASS), jnp.float32),
        pltpu.SemaphoreType.DMA,
        pltpu.SemaphoreType.DMA,
        pltpu.SemaphoreType.DMA,
    ],
)
def _scatter_exec(dsts_hbm, srcs_hbm, cnts_hbm, new_hbm, ema_ref,
                  dstl, srcl, cntv, db0, db1, sb0, sb1,
                  rb0, rb1, g0, g1, ssem):
  wid = lax.axis_index("s") * NC + lax.axis_index("c")
  pltpu.async_copy(cnts_hbm.at[wid], cntv, g0)
  pltpu.async_copy(dsts_hbm.at[wid], dstl, g1)
  pltpu.async_copy(srcs_hbm.at[wid], srcl, ssem)
  pltpu.make_async_copy(cnts_hbm.at[wid], cntv, g0).wait()
  pltpu.make_async_copy(dsts_hbm.at[wid], dstl, g1).wait()
  pltpu.make_async_copy(srcs_hbm.at[wid], srcl, ssem).wait()
  cnt = cntv[pl.ds(0, 16)][0]

  @pl.when(cnt > 0)
  def _():
    nch = (cnt + CHUNK - 1) // CHUNK
    dbs, sbs, rbs, gs = (db0, db1), (sb0, sb1), (rb0, rb1), (g0, g1)

    def prep_and_gather(c, s):
      for q in range(CHUNK // 16):
        dbs[s][pl.ds(q * 16, 16)] = dstl[pl.ds(c * CHUNK + q * 16, 16)]
        sbs[s][pl.ds(q * 16, 16)] = srcl[pl.ds(c * CHUNK + q * 16, 16)]
      pltpu.async_copy(new_hbm.at[sbs[s]], rbs[s], gs[s])

    prep_and_gather(0, 0)

    def outer(i, carry):
      for s in range(2):
        c = i * 2 + s

        @pl.when(c < nch)
        def _():
          pltpu.make_async_copy(new_hbm.at[sbs[s]], rbs[s], gs[s]).wait()

          @pl.when(c > 0)
          def _():
            pltpu.make_async_copy(rbs[s], ema_ref.at[dbs[s]], ssem).wait()

          @pl.when(c + 1 < nch)
          def _():
            prep_and_gather(c + 1, 1 - s)

          pltpu.async_copy(rbs[s], ema_ref.at[dbs[s]], ssem)

      return carry

    lax.fori_loop(0, (nch + 1) // 2, outer, 0)
    pltpu.make_async_copy(rb0, ema_ref.at[db0], ssem).wait()


def _dense_body(w_ref, t_ref, o0_ref, o1_ref, new_ref, ema_ref,
                emac_ref, tot_ref, acc):
  pid = pl.program_id(0)
  emac_ref[...] = ema_ref[...]
  o0 = o0_ref[...]
  o1 = o1_ref[...]
  new = new_ref[...]
  t = t_ref[0, 0, :]
  mask = (lax.broadcasted_iota(jnp.int32, o0.shape, 1) == t[:, None]).astype(
      jnp.float32)

  m0 = jnp.max(o0, axis=1, keepdims=True)
  lse0 = jnp.log(jnp.sum(jnp.exp(o0 - m0), axis=1)) + m0[:, 0]
  logp0t = jnp.sum(o0 * mask, axis=1) - lse0

  z = new * (1.0 / TEMP)
  mz = jnp.max(z, axis=1, keepdims=True)
  ez = jnp.exp(z - mz)
  alpha = jnp.sum(ez * mask, axis=1) / jnp.sum(ez, axis=1)
  w = w_ref[0]
  alphaw = w * alpha + (1.0 - w)

  m1 = jnp.max(o1, axis=1, keepdims=True)
  lse1 = jnp.log(jnp.sum(jnp.exp(o1 - m1), axis=1)) + m1[:, 0]
  t1 = jnp.sum(o1 * mask, axis=1) - lse1
  s1 = jnp.sum(o1, axis=1) - NUM_CLASS * lse1
  mterm = alphaw * t1 + (1.0 - alphaw) * (s1 - t1) * (1.0 / (NUM_CLASS - 1))

  part = jnp.sum(logp0t + LAM * mterm)

  @pl.when(pid == 0)
  def _():
    acc[0] = 0.0

  acc[0] += part

  @pl.when(pid == pl.num_programs(0) - 1)
  def _():
    tot_ref[0, 0] = -acc[0] / BATCH


_EB = 409600


def _dense(w, targets3d, outputs_0, outputs_1, new_rows, ema_flat):
  return pl.pallas_call(
      _dense_body,
      grid=(BATCH // BB,),
      in_specs=[
          pl.BlockSpec(memory_space=pltpu.SMEM),
          pl.BlockSpec((1, 1, BB), lambda i: (i, 0, 0)),
          pl.BlockSpec((BB, NUM_CLASS), lambda i: (i, 0)),
          pl.BlockSpec((BB, NUM_CLASS), lambda i: (i, 0)),
          pl.BlockSpec((BB, NUM_CLASS), lambda i: (i, 0)),
          pl.BlockSpec((_EB,), lambda i: (i,)),
      ],
      out_specs=[
          pl.BlockSpec((_EB,), lambda i: (i,)),
          pl.BlockSpec(memory_space=pltpu.SMEM),
      ],
      out_shape=[
          jax.ShapeDtypeStruct((NUM_EXAMP * NUM_CLASS,), jnp.float32),
          jax.ShapeDtypeStruct((1, 1), jnp.float32),
      ],
      scratch_shapes=[pltpu.SMEM((1,), jnp.float32)],
  )(w, targets3d, outputs_0, outputs_1, new_rows, ema_flat)


def kernel(outputs_0, outputs_1, targets, epoch, indexs, ema):
  w = jnp.minimum(jnp.float32(1.0),
                  jnp.asarray(epoch, jnp.float32) / WARMUP).reshape(1)
  targets3d = targets.reshape(BATCH // BB, 1, BB)

  new_rows = _gather_blend(indexs, ema, outputs_0)
  dsts, srcs, cnts = _plan(indexs)
  ema_copy_flat, tot = _dense(w, targets3d, outputs_0, outputs_1, new_rows,
                              ema.reshape(-1))

  ema_ref = jax.new_ref(ema_copy_flat.reshape(NUM_EXAMP, NUM_CLASS))
  _scatter_exec(dsts, srcs, cnts, new_rows, ema_ref)
  ema_new = ema_ref[...]

  return (tot[0, 0], ema_new)


# speedup vs baseline: 1.1427x; 1.0659x over previous
import functools

import jax
import jax.numpy as jnp
from jax import lax
from jax.experimental import pallas as pl
from jax.experimental.pallas import tpu as pltpu
from jax.experimental.pallas import tpu_sc as plsc

NUM_EXAMP = 100000
NUM_CLASS = 128
BATCH = 16384
LAM = 2.0
W_EMA = 0.7
TEMP = 0.33
WARMUP = 30

NC = 2
NS = 16
NW = NC * NS
RPT = NUM_EXAMP // NW
TS = 3136
LCAP = TS + 128
CHUNK = 128
BPW = BATCH // NW
BB = 1024

_mesh = plsc.VectorSubcoreMesh(core_axis_name="c", subcore_axis_name="s")
_sc_params = pltpu.CompilerParams(needs_layout_passes=False)


@functools.partial(
    pl.kernel,
    out_type=jax.ShapeDtypeStruct((BATCH, NUM_CLASS), jnp.float32),
    mesh=_mesh,
    compiler_params=_sc_params,
    scratch_types=[
        pltpu.VMEM((CHUNK,), jnp.int32),
        pltpu.VMEM((CHUNK,), jnp.int32),
        pltpu.VMEM((CHUNK,), jnp.int32),
        pltpu.VMEM((CHUNK, NUM_CLASS), jnp.float32),
        pltpu.VMEM((CHUNK, NUM_CLASS), jnp.float32),
        pltpu.VMEM((CHUNK, NUM_CLASS), jnp.float32),
        pltpu.VMEM((CHUNK, NUM_CLASS), jnp.float32),
        pltpu.VMEM((CHUNK, NUM_CLASS), jnp.float32),
        pltpu.VMEM((CHUNK, NUM_CLASS), jnp.float32),
        pltpu.SemaphoreType.DMA,
        pltpu.SemaphoreType.DMA,
        pltpu.SemaphoreType.DMA,
        pltpu.SemaphoreType.DMA,
        pltpu.SemaphoreType.DMA,
        pltpu.SemaphoreType.DMA,
        pltpu.SemaphoreType.DMA,
        pltpu.SemaphoreType.DMA,
        pltpu.SemaphoreType.DMA,
    ],
)
def _gather_blend(idx_hbm, ema_hbm, o0_hbm, new_hbm,
                  ib0, ib1, ib2, rb0, rb1, rb2, ob0, ob1, ob2,
                  g0, g1, g2, p0, p1, p2, w0, w1, w2):
  wid = lax.axis_index("s") * NC + lax.axis_index("c")
  base = wid * BPW
  ibs, rbs, obs = (ib0, ib1, ib2), (rb0, rb1, rb2), (ob0, ob1, ob2)
  gs, ps, ws = (g0, g1, g2), (p0, p1, p2), (w0, w1, w2)

  def idx_of(c):
    return base + c * CHUNK

  def issue(c, s):
    pltpu.sync_copy(idx_hbm.at[pl.ds(idx_of(c), CHUNK)], ibs[s])
    pltpu.async_copy(ema_hbm.at[ibs[s]], rbs[s], gs[s])
    pltpu.async_copy(o0_hbm.at[pl.ds(idx_of(c), CHUNK)], obs[s], ps[s])

  nch = BPW // CHUNK
  issue(0, 0)
  issue(1, 1)
  issue(2, 2)
  for c in range(nch):
    s = c % 3
    pltpu.make_async_copy(ema_hbm.at[ibs[s]], rbs[s], gs[s]).wait()
    pltpu.make_async_copy(o0_hbm.at[pl.ds(0, CHUNK)], obs[s], ps[s]).wait()

    def blend(r, carry, rb=rbs[s], ob=obs[s]):
      for q in range(NUM_CLASS // 16):
        sl = pl.ds(q * 16, 16)
        rb[r, sl] = W_EMA * rb[r, sl] + (1.0 - W_EMA) * ob[r, sl]
      return carry

    lax.fori_loop(0, CHUNK, blend, 0)
    pltpu.async_copy(rbs[s], new_hbm.at[pl.ds(idx_of(c), CHUNK)], ws[s])
    if c + 3 < nch:
      pltpu.make_async_copy(rbs[s], new_hbm.at[pl.ds(0, CHUNK)],
                            ws[s]).wait()
      issue(c + 3, s)
  for s in range(min(3, nch)):
    pltpu.make_async_copy(rbs[s], new_hbm.at[pl.ds(0, CHUNK)], ws[s]).wait()


@functools.partial(
    pl.kernel,
    out_type=(
        jax.ShapeDtypeStruct((NW, LCAP), jnp.int32),
        jax.ShapeDtypeStruct((NW, LCAP), jnp.int32),
        jax.ShapeDtypeStruct((NW, 16), jnp.int32),
    ),
    mesh=_mesh,
    compiler_params=_sc_params,
    scratch_types=[
        pltpu.VMEM((BATCH,), jnp.int32),
        pltpu.VMEM((16 * TS,), jnp.int32),
        pltpu.VMEM((LCAP,), jnp.int32),
        pltpu.VMEM((LCAP,), jnp.int32),
        pltpu.VMEM((16,), jnp.int32),
        pltpu.SemaphoreType.DMA,
    ],
)
def _plan(idx_hbm, dsts_hbm, srcs_hbm, cnts_hbm, idxv, tab, dstl, srcl, cntv,
          isem):
  wid = lax.axis_index("s") * NC + lax.axis_index("c")
  lo = wid * RPT
  pltpu.async_copy(idx_hbm, idxv, isem)

  lanes = lax.iota(jnp.int32, 16)
  neg1 = jnp.full((16,), -1, jnp.int32)

  @plsc.parallel_loop(0, TS, unroll=4)
  def _(i):
    tab[pl.ds(i * 16, 16)] = neg1

  pltpu.make_async_copy(idx_hbm, idxv, isem).wait()
  laneoff = lanes * TS - lo

  def p1_body(k, carry):
    for u in range(4):
      off = k * 64 + u * 16
      v = idxv[pl.ds(off, 16)]
      pos = lanes + off
      m = (v >= lo) & (v < lo + RPT)
      plsc.store_scatter(tab, [v + laneoff], pos, mask=m)
    return carry

  lax.fori_loop(0, BATCH // 64, p1_body, 0)

  @plsc.parallel_loop(0, TS // 16, unroll=2, carry=jnp.int32(0))
  def cnt(j, c_in):
    acc = tab[pl.ds(j * 16, 16)]
    for l in range(1, 16):
      acc = jnp.maximum(acc, tab[pl.ds(l * TS + j * 16, 16)])
    m = acc >= 0
    rows = lanes + (j * 16 + lo)
    plsc.store_compressed(dstl.at[pl.ds(c_in, 16)], rows, mask=m)
    plsc.store_compressed(srcl.at[pl.ds(c_in, 16)], acc, mask=m)
    return c_in + jnp.sum(jnp.where(m, 1, 0))

  @pl.when(cnt > 0)
  def _():
    dlast = dstl[pl.ds(cnt - 1, 16)]
    slast = srcl[pl.ds(cnt - 1, 16)]
    dpad = jnp.full((16,), dlast[0], jnp.int32)
    spad = jnp.full((16,), slast[0], jnp.int32)
    for q in range(CHUNK // 16):
      dstl[pl.ds(cnt + q * 16, 16)] = dpad
      srcl[pl.ds(cnt + q * 16, 16)] = spad

  cntv[...] = jnp.full((16,), cnt, jnp.int32)
  pltpu.sync_copy(dstl, dsts_hbm.at[wid])
  pltpu.sync_copy(srcl, srcs_hbm.at[wid])
  pltpu.sync_copy(cntv, cnts_hbm.at[wid])


@functools.partial(
    pl.kernel,
    out_type=(),
    mesh=_mesh,
    compiler_params=_sc_params,
    scratch_types=[
        pltpu.VMEM((LCAP,), jnp.int32),
        pltpu.VMEM((LCAP,), jnp.int32),
        pltpu.VMEM((16,), jnp.int32),
        pltpu.VMEM((CHUNK,), jnp.int32),
        pltpu.VMEM((CHUNK,), jnp.int32),
        pltpu.VMEM((CHUNK,), jnp.int32),
        pltpu.VMEM((CHUNK,), jnp.int32),
        pltpu.VMEM((CHUNK, NUM_CLASS), jnp.float32),
        pltpu.VMEM((CHUNK, NUM_CLASS), jnp.float32),
        pltpu.SemaphoreType.DMA,
        pltpu.SemaphoreType.DMA,
        pltpu.SemaphoreType.DMA,
    ],
)
def _scatter_exec(dsts_hbm, srcs_hbm, cnts_hbm, new_hbm, ema_ref,
                  dstl, srcl, cntv, db0, db1, sb0, sb1,
                  rb0, rb1, g0, g1, ssem):
  wid = lax.axis_index("s") * NC + lax.axis_index("c")
  pltpu.async_copy(cnts_hbm.at[wid], cntv, g0)
  pltpu.async_copy(dsts_hbm.at[wid], dstl, g1)
  pltpu.async_copy(srcs_hbm.at[wid], srcl, ssem)
  pltpu.make_async_copy(cnts_hbm.at[wid], cntv, g0).wait()
  pltpu.make_async_copy(dsts_hbm.at[wid], dstl, g1).wait()
  pltpu.make_async_copy(srcs_hbm.at[wid], srcl, ssem).wait()
  cnt = cntv[pl.ds(0, 16)][0]

  @pl.when(cnt > 0)
  def _():
    nch = (cnt + CHUNK - 1) // CHUNK
    dbs, sbs, rbs, gs = (db0, db1), (sb0, sb1), (rb0, rb1), (g0, g1)

    def prep_and_gather(c, s):
      for q in range(CHUNK // 16):
        dbs[s][pl.ds(q * 16, 16)] = dstl[pl.ds(c * CHUNK + q * 16, 16)]
        sbs[s][pl.ds(q * 16, 16)] = srcl[pl.ds(c * CHUNK + q * 16, 16)]
      pltpu.async_copy(new_hbm.at[sbs[s]], rbs[s], gs[s])

    prep_and_gather(0, 0)

    def outer(i, carry):
      for s in range(2):
        c = i * 2 + s

        @pl.when(c < nch)
        def _():
          pltpu.make_async_copy(new_hbm.at[sbs[s]], rbs[s], gs[s]).wait()

          @pl.when(c > 0)
          def _():
            pltpu.make_async_copy(rbs[s], ema_ref.at[dbs[s]], ssem).wait()

          @pl.when(c + 1 < nch)
          def _():
            prep_and_gather(c + 1, 1 - s)

          pltpu.async_copy(rbs[s], ema_ref.at[dbs[s]], ssem)

      return carry

    lax.fori_loop(0, (nch + 1) // 2, outer, 0)
    pltpu.make_async_copy(rb0, ema_ref.at[db0], ssem).wait()


def _dense_body(w_ref, t_ref, o0_ref, o1_ref, new_ref, ema_ref,
                emac_ref, tot_ref, acc):
  pid = pl.program_id(0)
  emac_ref[...] = ema_ref[...]
  o0 = o0_ref[...]
  o1 = o1_ref[...]
  new = new_ref[...]
  t = t_ref[0, 0, :]
  mask = (lax.broadcasted_iota(jnp.int32, o0.shape, 1) == t[:, None]).astype(
      jnp.float32)

  m0 = jnp.max(o0, axis=1, keepdims=True)
  lse0 = jnp.log(jnp.sum(jnp.exp(o0 - m0), axis=1)) + m0[:, 0]
  logp0t = jnp.sum(o0 * mask, axis=1) - lse0

  z = new * (1.0 / TEMP)
  mz = jnp.max(z, axis=1, keepdims=True)
  ez = jnp.exp(z - mz)
  alpha = jnp.sum(ez * mask, axis=1) / jnp.sum(ez, axis=1)
  w = w_ref[0]
  alphaw = w * alpha + (1.0 - w)

  m1 = jnp.max(o1, axis=1, keepdims=True)
  lse1 = jnp.log(jnp.sum(jnp.exp(o1 - m1), axis=1)) + m1[:, 0]
  t1 = jnp.sum(o1 * mask, axis=1) - lse1
  s1 = jnp.sum(o1, axis=1) - NUM_CLASS * lse1
  mterm = alphaw * t1 + (1.0 - alphaw) * (s1 - t1) * (1.0 / (NUM_CLASS - 1))

  part = jnp.sum(logp0t + LAM * mterm)

  @pl.when(pid == 0)
  def _():
    acc[0] = 0.0

  acc[0] += part

  @pl.when(pid == pl.num_programs(0) - 1)
  def _():
    tot_ref[0, 0] = -acc[0] / BATCH


_EB = 819200


def _dense(w, targets3d, outputs_0, outputs_1, new_rows, ema_flat):
  return pl.pallas_call(
      _dense_body,
      grid=(BATCH // BB,),
      in_specs=[
          pl.BlockSpec(memory_space=pltpu.SMEM),
          pl.BlockSpec((1, 1, BB), lambda i: (i, 0, 0)),
          pl.BlockSpec((BB, NUM_CLASS), lambda i: (i, 0)),
          pl.BlockSpec((BB, NUM_CLASS), lambda i: (i, 0)),
          pl.BlockSpec((BB, NUM_CLASS), lambda i: (i, 0)),
          pl.BlockSpec((_EB,), lambda i: (i,)),
      ],
      out_specs=[
          pl.BlockSpec((_EB,), lambda i: (i,)),
          pl.BlockSpec(memory_space=pltpu.SMEM),
      ],
      out_shape=[
          jax.ShapeDtypeStruct((NUM_EXAMP * NUM_CLASS,), jnp.float32),
          jax.ShapeDtypeStruct((1, 1), jnp.float32),
      ],
      scratch_shapes=[pltpu.SMEM((1,), jnp.float32)],
  )(w, targets3d, outputs_0, outputs_1, new_rows, ema_flat)


def kernel(outputs_0, outputs_1, targets, epoch, indexs, ema):
  w = jnp.minimum(jnp.float32(1.0),
                  jnp.asarray(epoch, jnp.float32) / WARMUP).reshape(1)
  targets3d = targets.reshape(BATCH // BB, 1, BB)

  new_rows = _gather_blend(indexs, ema, outputs_0)
  dsts, srcs, cnts = _plan(indexs)
  ema_copy_flat, tot = _dense(w, targets3d, outputs_0, outputs_1, new_rows,
                              ema.reshape(-1))

  ema_ref = jax.new_ref(ema_copy_flat.reshape(NUM_EXAMP, NUM_CLASS))
  _scatter_exec(dsts, srcs, cnts, new_rows, ema_ref)
  ema_new = ema_ref[...]

  return (tot[0, 0], ema_new)


# speedup vs baseline: 1.2071x; 1.0564x over previous
import functools

import jax
import jax.numpy as jnp
from jax import lax
from jax.experimental import pallas as pl
from jax.experimental.pallas import tpu as pltpu
from jax.experimental.pallas import tpu_sc as plsc

NUM_EXAMP = 100000
NUM_CLASS = 128
BATCH = 16384
LAM = 2.0
W_EMA = 0.7
TEMP = 0.33
WARMUP = 30

NC = 2
NS = 16
NW = NC * NS
RPT = NUM_EXAMP // NW
TS = 3136
LCAP = TS + 128
CHUNK = 128
BPW = BATCH // NW
BB = 2048

_mesh = plsc.VectorSubcoreMesh(core_axis_name="c", subcore_axis_name="s")
_sc_params = pltpu.CompilerParams(needs_layout_passes=False)


@functools.partial(
    pl.kernel,
    out_type=jax.ShapeDtypeStruct((BATCH, NUM_CLASS), jnp.float32),
    mesh=_mesh,
    compiler_params=_sc_params,
    scratch_types=[
        pltpu.VMEM((CHUNK,), jnp.int32),
        pltpu.VMEM((CHUNK,), jnp.int32),
        pltpu.VMEM((CHUNK,), jnp.int32),
        pltpu.VMEM((CHUNK, NUM_CLASS), jnp.float32),
        pltpu.VMEM((CHUNK, NUM_CLASS), jnp.float32),
        pltpu.VMEM((CHUNK, NUM_CLASS), jnp.float32),
        pltpu.VMEM((CHUNK, NUM_CLASS), jnp.float32),
        pltpu.VMEM((CHUNK, NUM_CLASS), jnp.float32),
        pltpu.VMEM((CHUNK, NUM_CLASS), jnp.float32),
        pltpu.SemaphoreType.DMA,
        pltpu.SemaphoreType.DMA,
        pltpu.SemaphoreType.DMA,
        pltpu.SemaphoreType.DMA,
        pltpu.SemaphoreType.DMA,
        pltpu.SemaphoreType.DMA,
        pltpu.SemaphoreType.DMA,
        pltpu.SemaphoreType.DMA,
        pltpu.SemaphoreType.DMA,
    ],
)
def _gather_blend(idx_hbm, ema_hbm, o0_hbm, new_hbm,
                  ib0, ib1, ib2, rb0, rb1, rb2, ob0, ob1, ob2,
                  g0, g1, g2, p0, p1, p2, w0, w1, w2):
  wid = lax.axis_index("s") * NC + lax.axis_index("c")
  base = wid * BPW
  ibs, rbs, obs = (ib0, ib1, ib2), (rb0, rb1, rb2), (ob0, ob1, ob2)
  gs, ps, ws = (g0, g1, g2), (p0, p1, p2), (w0, w1, w2)

  def idx_of(c):
    return base + c * CHUNK

  def issue(c, s):
    pltpu.sync_copy(idx_hbm.at[pl.ds(idx_of(c), CHUNK)], ibs[s])
    pltpu.async_copy(ema_hbm.at[ibs[s]], rbs[s], gs[s])
    pltpu.async_copy(o0_hbm.at[pl.ds(idx_of(c), CHUNK)], obs[s], ps[s])

  nch = BPW // CHUNK
  issue(0, 0)
  issue(1, 1)
  issue(2, 2)
  for c in range(nch):
    s = c % 3
    pltpu.make_async_copy(ema_hbm.at[ibs[s]], rbs[s], gs[s]).wait()
    pltpu.make_async_copy(o0_hbm.at[pl.ds(0, CHUNK)], obs[s], ps[s]).wait()

    def blend(r, carry, rb=rbs[s], ob=obs[s]):
      for q in range(NUM_CLASS // 16):
        sl = pl.ds(q * 16, 16)
        rb[r, sl] = W_EMA * rb[r, sl] + (1.0 - W_EMA) * ob[r, sl]
      return carry

    lax.fori_loop(0, CHUNK, blend, 0)
    pltpu.async_copy(rbs[s], new_hbm.at[pl.ds(idx_of(c), CHUNK)], ws[s])
    if c + 3 < nch:
      pltpu.make_async_copy(rbs[s], new_hbm.at[pl.ds(0, CHUNK)],
                            ws[s]).wait()
      issue(c + 3, s)
  for s in range(min(3, nch)):
    pltpu.make_async_copy(rbs[s], new_hbm.at[pl.ds(0, CHUNK)], ws[s]).wait()


@functools.partial(
    pl.kernel,
    out_type=(
        jax.ShapeDtypeStruct((NW, LCAP), jnp.int32),
        jax.ShapeDtypeStruct((NW, LCAP), jnp.int32),
        jax.ShapeDtypeStruct((NW, 16), jnp.int32),
    ),
    mesh=_mesh,
    compiler_params=_sc_params,
    scratch_types=[
        pltpu.VMEM((BATCH,), jnp.int32),
        pltpu.VMEM((16 * TS,), jnp.int32),
        pltpu.VMEM((LCAP,), jnp.int32),
        pltpu.VMEM((LCAP,), jnp.int32),
        pltpu.VMEM((16,), jnp.int32),
        pltpu.SemaphoreType.DMA,
    ],
)
def _plan(idx_hbm, dsts_hbm, srcs_hbm, cnts_hbm, idxv, tab, dstl, srcl, cntv,
          isem):
  wid = lax.axis_index("s") * NC + lax.axis_index("c")
  lo = wid * RPT
  pltpu.async_copy(idx_hbm, idxv, isem)

  lanes = lax.iota(jnp.int32, 16)
  neg1 = jnp.full((16,), -1, jnp.int32)

  @plsc.parallel_loop(0, TS, unroll=4)
  def _(i):
    tab[pl.ds(i * 16, 16)] = neg1

  pltpu.make_async_copy(idx_hbm, idxv, isem).wait()
  laneoff = lanes * TS - lo

  def p1_body(k, carry):
    for u in range(4):
      off = k * 64 + u * 16
      v = idxv[pl.ds(off, 16)]
      pos = lanes + off
      m = (v >= lo) & (v < lo + RPT)
      plsc.store_scatter(tab, [v + laneoff], pos, mask=m)
    return carry

  lax.fori_loop(0, BATCH // 64, p1_body, 0)

  @plsc.parallel_loop(0, TS // 16, unroll=2, carry=jnp.int32(0))
  def cnt(j, c_in):
    acc = tab[pl.ds(j * 16, 16)]
    for l in range(1, 16):
      acc = jnp.maximum(acc, tab[pl.ds(l * TS + j * 16, 16)])
    m = acc >= 0
    rows = lanes + (j * 16 + lo)
    plsc.store_compressed(dstl.at[pl.ds(c_in, 16)], rows, mask=m)
    plsc.store_compressed(srcl.at[pl.ds(c_in, 16)], acc, mask=m)
    return c_in + jnp.sum(jnp.where(m, 1, 0))

  @pl.when(cnt > 0)
  def _():
    dlast = dstl[pl.ds(cnt - 1, 16)]
    slast = srcl[pl.ds(cnt - 1, 16)]
    dpad = jnp.full((16,), dlast[0], jnp.int32)
    spad = jnp.full((16,), slast[0], jnp.int32)
    for q in range(CHUNK // 16):
      dstl[pl.ds(cnt + q * 16, 16)] = dpad
      srcl[pl.ds(cnt + q * 16, 16)] = spad

  cntv[...] = jnp.full((16,), cnt, jnp.int32)
  pltpu.sync_copy(dstl, dsts_hbm.at[wid])
  pltpu.sync_copy(srcl, srcs_hbm.at[wid])
  pltpu.sync_copy(cntv, cnts_hbm.at[wid])


@functools.partial(
    pl.kernel,
    out_type=(),
    mesh=_mesh,
    compiler_params=_sc_params,
    scratch_types=[
        pltpu.VMEM((LCAP,), jnp.int32),
        pltpu.VMEM((LCAP,), jnp.int32),
        pltpu.VMEM((16,), jnp.int32),
        pltpu.VMEM((CHUNK,), jnp.int32),
        pltpu.VMEM((CHUNK,), jnp.int32),
        pltpu.VMEM((CHUNK,), jnp.int32),
        pltpu.VMEM((CHUNK,), jnp.int32),
        pltpu.VMEM((CHUNK, NUM_CLASS), jnp.float32),
        pltpu.VMEM((CHUNK, NUM_CLASS), jnp.float32),
        pltpu.SemaphoreType.DMA,
        pltpu.SemaphoreType.DMA,
        pltpu.SemaphoreType.DMA,
    ],
)
def _scatter_exec(dsts_hbm, srcs_hbm, cnts_hbm, new_hbm, ema_ref,
                  dstl, srcl, cntv, db0, db1, sb0, sb1,
                  rb0, rb1, g0, g1, ssem):
  wid = lax.axis_index("s") * NC + lax.axis_index("c")
  pltpu.async_copy(cnts_hbm.at[wid], cntv, g0)
  pltpu.async_copy(dsts_hbm.at[wid], dstl, g1)
  pltpu.async_copy(srcs_hbm.at[wid], srcl, ssem)
  pltpu.make_async_copy(cnts_hbm.at[wid], cntv, g0).wait()
  pltpu.make_async_copy(dsts_hbm.at[wid], dstl, g1).wait()
  pltpu.make_async_copy(srcs_hbm.at[wid], srcl, ssem).wait()
  cnt = cntv[pl.ds(0, 16)][0]

  @pl.when(cnt > 0)
  def _():
    nch = (cnt + CHUNK - 1) // CHUNK
    dbs, sbs, rbs, gs = (db0, db1), (sb0, sb1), (rb0, rb1), (g0, g1)

    def prep_and_gather(c, s):
      for q in range(CHUNK // 16):
        dbs[s][pl.ds(q * 16, 16)] = dstl[pl.ds(c * CHUNK + q * 16, 16)]
        sbs[s][pl.ds(q * 16, 16)] = srcl[pl.ds(c * CHUNK + q * 16, 16)]
      pltpu.async_copy(new_hbm.at[sbs[s]], rbs[s], gs[s])

    prep_and_gather(0, 0)

    def outer(i, carry):
      for s in range(2):
        c = i * 2 + s

        @pl.when(c < nch)
        def _():
          pltpu.make_async_copy(new_hbm.at[sbs[s]], rbs[s], gs[s]).wait()

          @pl.when(c > 0)
          def _():
            pltpu.make_async_copy(rbs[s], ema_ref.at[dbs[s]], ssem).wait()

          @pl.when(c + 1 < nch)
          def _():
            prep_and_gather(c + 1, 1 - s)

          pltpu.async_copy(rbs[s], ema_ref.at[dbs[s]], ssem)

      return carry

    lax.fori_loop(0, (nch + 1) // 2, outer, 0)
    pltpu.make_async_copy(rb0, ema_ref.at[db0], ssem).wait()


def _dense_body(w_ref, t_ref, o0_ref, o1_ref, new_ref, ema_ref,
                emac_ref, tot_ref, acc):
  pid = pl.program_id(0)
  emac_ref[...] = ema_ref[...]
  o0 = o0_ref[...]
  o1 = o1_ref[...]
  new = new_ref[...]
  t = t_ref[0, 0, :]
  mask = (lax.broadcasted_iota(jnp.int32, o0.shape, 1) == t[:, None]).astype(
      jnp.float32)

  m0 = jnp.max(o0, axis=1, keepdims=True)
  lse0 = jnp.log(jnp.sum(jnp.exp(o0 - m0), axis=1)) + m0[:, 0]
  logp0t = jnp.sum(o0 * mask, axis=1) - lse0

  z = new * (1.0 / TEMP)
  mz = jnp.max(z, axis=1, keepdims=True)
  ez = jnp.exp(z - mz)
  alpha = jnp.sum(ez * mask, axis=1) / jnp.sum(ez, axis=1)
  w = w_ref[0]
  alphaw = w * alpha + (1.0 - w)

  m1 = jnp.max(o1, axis=1, keepdims=True)
  lse1 = jnp.log(jnp.sum(jnp.exp(o1 - m1), axis=1)) + m1[:, 0]
  t1 = jnp.sum(o1 * mask, axis=1) - lse1
  s1 = jnp.sum(o1, axis=1) - NUM_CLASS * lse1
  mterm = alphaw * t1 + (1.0 - alphaw) * (s1 - t1) * (1.0 / (NUM_CLASS - 1))

  part = jnp.sum(logp0t + LAM * mterm)

  @pl.when(pid == 0)
  def _():
    acc[0] = 0.0

  acc[0] += part

  @pl.when(pid == pl.num_programs(0) - 1)
  def _():
    tot_ref[0, 0] = -acc[0] / BATCH


_EB = 1638400


def _dense(w, targets3d, outputs_0, outputs_1, new_rows, ema_flat):
  return pl.pallas_call(
      _dense_body,
      grid=(BATCH // BB,),
      in_specs=[
          pl.BlockSpec(memory_space=pltpu.SMEM),
          pl.BlockSpec((1, 1, BB), lambda i: (i, 0, 0)),
          pl.BlockSpec((BB, NUM_CLASS), lambda i: (i, 0)),
          pl.BlockSpec((BB, NUM_CLASS), lambda i: (i, 0)),
          pl.BlockSpec((BB, NUM_CLASS), lambda i: (i, 0)),
          pl.BlockSpec((_EB,), lambda i: (i,)),
      ],
      out_specs=[
          pl.BlockSpec((_EB,), lambda i: (i,)),
          pl.BlockSpec(memory_space=pltpu.SMEM),
      ],
      out_shape=[
          jax.ShapeDtypeStruct((NUM_EXAMP * NUM_CLASS,), jnp.float32),
          jax.ShapeDtypeStruct((1, 1), jnp.float32),
      ],
      scratch_shapes=[pltpu.SMEM((1,), jnp.float32)],
  )(w, targets3d, outputs_0, outputs_1, new_rows, ema_flat)


def kernel(outputs_0, outputs_1, targets, epoch, indexs, ema):
  w = jnp.minimum(jnp.float32(1.0),
                  jnp.asarray(epoch, jnp.float32) / WARMUP).reshape(1)
  targets3d = targets.reshape(BATCH // BB, 1, BB)

  new_rows = _gather_blend(indexs, ema, outputs_0)
  dsts, srcs, cnts = _plan(indexs)
  ema_copy_flat, tot = _dense(w, targets3d, outputs_0, outputs_1, new_rows,
                              ema.reshape(-1))

  ema_ref = jax.new_ref(ema_copy_flat.reshape(NUM_EXAMP, NUM_CLASS))
  _scatter_exec(dsts, srcs, cnts, new_rows, ema_ref)
  ema_new = ema_ref[...]

  return (tot[0, 0], ema_new)
